# Initial kernel scaffold; baseline (speedup 1.0000x reference)
import jax
import jax.numpy as jnp
from jax.experimental import pallas as pl


def kernel(x, edge_index, W1, b1, W2, b2):
    raise NotImplementedError("write your pallas kernel here")


# speedup vs baseline: 53.6199x; 53.6199x over previous
import functools

import jax
import jax.numpy as jnp
from jax import lax
from jax.experimental import pallas as pl
from jax.experimental.pallas import tpu as pltpu
from jax.experimental.pallas import tpu_sc as plsc

NC = 2
NS = 16
NW = NC * NS
LANES = 16
D_HID = 16


def _round_up(v, m):
    return (v + m - 1) // m * m


def _make_sc_deg(n_pad, per_w):
    rows_per_sub = n_pad // NS
    mesh = plsc.VectorSubcoreMesh(core_axis_name="c", subcore_axis_name="s")

    @functools.partial(
        pl.kernel,
        out_type=jax.ShapeDtypeStruct((NC, n_pad), jnp.float32),
        mesh=mesh,
        scratch_types=[
            pltpu.VMEM_SHARED((n_pad,), jnp.float32),
            pltpu.VMEM((8, 128), jnp.int32),
            pltpu.VMEM((128,), jnp.float32),
            pltpu.VMEM((128,), jnp.float32),
        ],
    )
    def deg_kernel(dst_hbm, degp_hbm, acc, idx, ones_b, zbuf):
        cid = lax.axis_index("c")
        sid = lax.axis_index("s")
        wid = cid * NS + sid

        def fill(i, _):
            ones_b[pl.ds(i * LANES, LANES)] = jnp.ones((LANES,), jnp.float32)
            zbuf[pl.ds(i * LANES, LANES)] = jnp.zeros((LANES,), jnp.float32)
            return 0

        lax.fori_loop(0, 128 // LANES, fill, 0)

        base = sid * rows_per_sub

        def zloop(i, _):
            pltpu.sync_copy(zbuf, acc.at[pl.ds(base + i * 128, 128)])
            return 0

        lax.fori_loop(0, rows_per_sub // 128, zloop, 0)
        plsc.subcore_barrier()

        row0 = wid * (per_w // 128)

        def eloop(j, _):
            pltpu.sync_copy(dst_hbm.at[pl.ds(row0 + j * 8, 8)], idx)
            for k in range(8):
                pltpu.sync_copy(ones_b, acc.at[idx.at[k]], add=True)
            return 0

        lax.fori_loop(0, per_w // 1024, eloop, 0)
        plsc.subcore_barrier()
        pltpu.sync_copy(
            acc.at[pl.ds(base, rows_per_sub)],
            degp_hbm.at[cid, pl.ds(base, rows_per_sub)],
        )

    return deg_kernel


def _make_sc_msgpass(n_pad, per_w):
    rows_per_sub = n_pad // NS
    mesh = plsc.VectorSubcoreMesh(core_axis_name="c", subcore_axis_name="s")

    @functools.partial(
        pl.kernel,
        out_type=jax.ShapeDtypeStruct((NC, n_pad, D_HID), jnp.float32),
        mesh=mesh,
        scratch_types=[
            pltpu.VMEM_SHARED((n_pad, D_HID), jnp.float32),
            pltpu.VMEM((8, 128), jnp.int32),
            pltpu.VMEM((8, 128), jnp.int32),
            pltpu.VMEM((8, 128, D_HID), jnp.float32),
            pltpu.VMEM((128, D_HID), jnp.float32),
            pltpu.SemaphoreType.DMA,
        ],
        compiler_params=pltpu.CompilerParams(use_tc_tiling_on_sc=False),
    )
    def mp_kernel(hs_hbm, src_hbm, dst_hbm, aggp_hbm, acc, idx_s, idx_d, rows,
                  zbuf, sem):
        cid = lax.axis_index("c")
        sid = lax.axis_index("s")
        wid = cid * NS + sid

        def fillz(i, _):
            zbuf[i, :] = jnp.zeros((LANES,), jnp.float32)
            return 0

        lax.fori_loop(0, 128, fillz, 0)

        base = sid * rows_per_sub

        def zloop(i, _):
            pltpu.sync_copy(zbuf, acc.at[pl.ds(base + i * 128, 128)])
            return 0

        lax.fori_loop(0, rows_per_sub // 128, zloop, 0)
        plsc.subcore_barrier()

        row0 = wid * (per_w // 128)

        def eloop(j, _):
            r = row0 + j * 8
            pltpu.sync_copy(src_hbm.at[pl.ds(r, 8)], idx_s)
            pltpu.sync_copy(dst_hbm.at[pl.ds(r, 8)], idx_d)
            cps = [
                pltpu.async_copy(hs_hbm.at[idx_s.at[k]], rows.at[k], sem)
                for k in range(8)
            ]
            for cp in cps:
                cp.wait()
            for k in range(8):
                pltpu.sync_copy(rows.at[k], acc.at[idx_d.at[k]], add=True)
            return 0

        lax.fori_loop(0, per_w // 1024, eloop, 0)
        plsc.subcore_barrier()
        pltpu.sync_copy(
            acc.at[pl.ds(base, rows_per_sub)],
            aggp_hbm.at[cid, pl.ds(base, rows_per_sub)],
        )

    return mp_kernel


_BLK = 1024


def _tc_a_body(x_ref, w1_ref, degp_ref, hs1_ref):
    h = jnp.dot(x_ref[...], w1_ref[...], preferred_element_type=jnp.float32)
    deg = degp_ref[0] + degp_ref[1] + 1.0
    dinv = lax.rsqrt(deg)
    hs1_ref[...] = h * dinv[:, None]


def _tc_b_body(aggp_ref, hs1_ref, degp_ref, b1_ref, w2_ref, hs2_ref):
    deg = degp_ref[0] + degp_ref[1] + 1.0
    dinv = lax.rsqrt(deg)
    agg = aggp_ref[0] + aggp_ref[1]
    z1 = dinv[:, None] * (agg + hs1_ref[...]) + b1_ref[...]
    a1 = jnp.maximum(z1, 0.0)
    h2 = jnp.dot(a1, w2_ref[...], preferred_element_type=jnp.float32)
    hs2_ref[...] = h2 * dinv[:, None]


def _tc_c_body(aggp_ref, hs2_ref, degp_ref, b2_ref, out_ref):
    deg = degp_ref[0] + degp_ref[1] + 1.0
    dinv = lax.rsqrt(deg)
    z = dinv[:, None] * (aggp_ref[0] + aggp_ref[1] + hs2_ref[...]) + b2_ref[...]
    m = jnp.max(z, axis=1, keepdims=True)
    lse = jnp.log(jnp.sum(jnp.exp(z - m), axis=1, keepdims=True)) + m
    out_ref[...] = z - lse


def kernel(x, edge_index, W1, b1, W2, b2):
    n, d_in = x.shape
    e = edge_index.shape[1]
    d_hid = W1.shape[1]

    n_pad = _round_up(n, _BLK)
    per_w = _round_up((e + NW - 1) // NW, _BLK)
    e_pad = per_w * NW

    src = edge_index[0].astype(jnp.int32)
    dst = edge_index[1].astype(jnp.int32)
    npad_e = e_pad - e
    if npad_e:
        spread = max(n_pad - n, 1)
        pad_idx = n + (jnp.arange(npad_e, dtype=jnp.int32) % spread)
        src = jnp.concatenate([src, pad_idx])
        dst = jnp.concatenate([dst, pad_idx])
    src2d = src.reshape(e_pad // 128, 128)
    dst2d = dst.reshape(e_pad // 128, 128)

    sc_deg = _make_sc_deg(n_pad, per_w)
    sc_mp = _make_sc_msgpass(n_pad, per_w)

    degp = sc_deg(dst2d)

    grid = (n_pad // _BLK,)
    row_spec = pl.BlockSpec((_BLK, d_hid), lambda i: (i, 0))
    degp_spec = pl.BlockSpec((NC, _BLK), lambda i: (0, i))
    aggp_spec = pl.BlockSpec((NC, _BLK, d_hid), lambda i: (0, i, 0))
    bias_spec = pl.BlockSpec((1, d_hid), lambda i: (0, 0))

    hs1 = pl.pallas_call(
        _tc_a_body,
        grid=grid,
        in_specs=[
            pl.BlockSpec((_BLK, d_in), lambda i: (i, 0)),
            pl.BlockSpec((d_in, d_hid), lambda i: (0, 0)),
            degp_spec,
        ],
        out_specs=row_spec,
        out_shape=jax.ShapeDtypeStruct((n_pad, d_hid), jnp.float32),
    )(x, W1, degp)

    agg1 = sc_mp(hs1, src2d, dst2d)

    hs2 = pl.pallas_call(
        _tc_b_body,
        grid=grid,
        in_specs=[
            aggp_spec,
            row_spec,
            degp_spec,
            bias_spec,
            pl.BlockSpec((d_hid, d_hid), lambda i: (0, 0)),
        ],
        out_specs=row_spec,
        out_shape=jax.ShapeDtypeStruct((n_pad, d_hid), jnp.float32),
    )(agg1, hs1, degp, b1.reshape(1, d_hid), W2)

    agg2 = sc_mp(hs2, src2d, dst2d)

    out = pl.pallas_call(
        _tc_c_body,
        grid=grid,
        in_specs=[aggp_spec, row_spec, degp_spec, bias_spec],
        out_specs=row_spec,
        out_shape=jax.ShapeDtypeStruct((n_pad, d_hid), jnp.float32),
    )(agg2, hs2, degp, b2.reshape(1, d_hid))

    return out[:n]


# speedup vs baseline: 66.2342x; 1.2353x over previous
import functools

import jax
import jax.numpy as jnp
from jax import lax
from jax.experimental import pallas as pl
from jax.experimental.pallas import tpu as pltpu
from jax.experimental.pallas import tpu_sc as plsc

NC = 2
NS = 16
NW = NC * NS
LANES = 16
D_HID = 16
NB = 4


def _round_up(v, m):
    return (v + m - 1) // m * m


def _make_sc_deg(n_pad, per_w):
    rows_per_sub = n_pad // NS
    mesh = plsc.VectorSubcoreMesh(core_axis_name="c", subcore_axis_name="s")

    @functools.partial(
        pl.kernel,
        out_type=jax.ShapeDtypeStruct((NC, n_pad), jnp.float32),
        mesh=mesh,
        scratch_types=[
            pltpu.VMEM_SHARED((n_pad,), jnp.float32),
            pltpu.VMEM((8, 128), jnp.int32),
            pltpu.VMEM((128,), jnp.float32),
            pltpu.VMEM((128,), jnp.float32),
        ],
    )
    def deg_kernel(dst_hbm, degp_hbm, acc, idx, ones_b, zbuf):
        cid = lax.axis_index("c")
        sid = lax.axis_index("s")
        wid = cid * NS + sid

        def fill(i, _):
            ones_b[pl.ds(i * LANES, LANES)] = jnp.ones((LANES,), jnp.float32)
            zbuf[pl.ds(i * LANES, LANES)] = jnp.zeros((LANES,), jnp.float32)
            return 0

        lax.fori_loop(0, 128 // LANES, fill, 0)

        base = sid * rows_per_sub

        def zloop(i, _):
            pltpu.sync_copy(zbuf, acc.at[pl.ds(base + i * 128, 128)])
            return 0

        lax.fori_loop(0, rows_per_sub // 128, zloop, 0)
        plsc.subcore_barrier()

        row0 = wid * (per_w // 128)

        def eloop(j, _):
            pltpu.sync_copy(dst_hbm.at[pl.ds(row0 + j * 8, 8)], idx)
            for k in range(8):
                pltpu.sync_copy(ones_b, acc.at[idx.at[k]], add=True)
            return 0

        lax.fori_loop(0, per_w // 1024, eloop, 0)
        plsc.subcore_barrier()
        pltpu.sync_copy(
            acc.at[pl.ds(base, rows_per_sub)],
            degp_hbm.at[cid, pl.ds(base, rows_per_sub)],
        )

    return deg_kernel


def _make_sc_msgpass(n_pad, per_w):
    rows_per_sub = n_pad // NS
    mesh = plsc.VectorSubcoreMesh(core_axis_name="c", subcore_axis_name="s")

    n_chunks = per_w // (NB * 128)

    @functools.partial(
        pl.kernel,
        out_type=jax.ShapeDtypeStruct((NC, n_pad, D_HID), jnp.float32),
        mesh=mesh,
        scratch_types=[
            pltpu.VMEM_SHARED((n_pad, D_HID), jnp.float32),
            pltpu.VMEM((2, NB, 128), jnp.int32),
            pltpu.VMEM((2, NB, 128), jnp.int32),
            pltpu.VMEM((2, NB, 128, D_HID), jnp.float32),
            pltpu.VMEM((128, D_HID), jnp.float32),
            pltpu.SemaphoreType.DMA,
            pltpu.SemaphoreType.DMA,
        ],
        compiler_params=pltpu.CompilerParams(use_tc_tiling_on_sc=False),
    )
    def mp_kernel(hs_hbm, src_hbm, dst_hbm, aggp_hbm, acc, idx_s, idx_d, rows,
                  zbuf, sem_g, sem_i):
        cid = lax.axis_index("c")
        sid = lax.axis_index("s")
        wid = cid * NS + sid

        def fillz(i, _):
            zbuf[i, :] = jnp.zeros((LANES,), jnp.float32)
            return 0

        lax.fori_loop(0, 128, fillz, 0)

        base = sid * rows_per_sub

        def zloop(i, _):
            pltpu.sync_copy(zbuf, acc.at[pl.ds(base + i * 128, 128)])
            return 0

        lax.fori_loop(0, rows_per_sub // 128, zloop, 0)
        plsc.subcore_barrier()

        row0 = wid * (per_w // 128)

        def fire_idx(j, p):
            r = row0 + j * NB
            pltpu.async_copy(src_hbm.at[pl.ds(r, NB)], idx_s.at[p], sem_i)
            pltpu.async_copy(dst_hbm.at[pl.ds(r, NB)], idx_d.at[p], sem_i)

        def wait_idx(j, p):
            r = row0 + j * NB
            pltpu.make_async_copy(src_hbm.at[pl.ds(r, NB)], idx_s.at[p],
                                  sem_i).wait()
            pltpu.make_async_copy(dst_hbm.at[pl.ds(r, NB)], idx_d.at[p],
                                  sem_i).wait()

        def fire_gathers(p):
            for k in range(NB):
                pltpu.async_copy(hs_hbm.at[idx_s.at[p, k]], rows.at[p, k],
                                 sem_g)

        def wait_gathers(p):
            for k in range(NB):
                pltpu.make_async_copy(hs_hbm.at[idx_s.at[p, k]],
                                      rows.at[p, k], sem_g).wait()

        def scatter(p):
            for k in range(NB):
                pltpu.sync_copy(rows.at[p, k], acc.at[idx_d.at[p, k]],
                                add=True)

        wait_idx_0 = pltpu.async_copy(src_hbm.at[pl.ds(row0, NB)],
                                      idx_s.at[0], sem_i)
        wait_idx_0d = pltpu.async_copy(dst_hbm.at[pl.ds(row0, NB)],
                                       idx_d.at[0], sem_i)
        fire_idx(1, 1)
        wait_idx_0.wait()
        wait_idx_0d.wait()
        fire_gathers(0)

        def eloop(j, _):
            p = lax.rem(j, 2)
            wait_gathers(p)

            @pl.when(j < n_chunks - 1)
            def _():
                wait_idx(j + 1, 1 - p)
                fire_gathers(1 - p)

            scatter(p)

            @pl.when(j < n_chunks - 2)
            def _():
                fire_idx(j + 2, p)

            return 0

        lax.fori_loop(0, n_chunks, eloop, 0)
        plsc.subcore_barrier()
        pltpu.sync_copy(
            acc.at[pl.ds(base, rows_per_sub)],
            aggp_hbm.at[cid, pl.ds(base, rows_per_sub)],
        )

    return mp_kernel


_BLK = 1024


def _tc_a_body(x_ref, w1_ref, degp_ref, hs1_ref):
    h = jnp.dot(x_ref[...], w1_ref[...], preferred_element_type=jnp.float32)
    deg = degp_ref[0] + degp_ref[1] + 1.0
    dinv = lax.rsqrt(deg)
    hs1_ref[...] = h * dinv[:, None]


def _tc_b_body(aggp_ref, hs1_ref, degp_ref, b1_ref, w2_ref, hs2_ref):
    deg = degp_ref[0] + degp_ref[1] + 1.0
    dinv = lax.rsqrt(deg)
    agg = aggp_ref[0] + aggp_ref[1]
    z1 = dinv[:, None] * (agg + hs1_ref[...]) + b1_ref[...]
    a1 = jnp.maximum(z1, 0.0)
    h2 = jnp.dot(a1, w2_ref[...], preferred_element_type=jnp.float32)
    hs2_ref[...] = h2 * dinv[:, None]


def _tc_c_body(aggp_ref, hs2_ref, degp_ref, b2_ref, out_ref):
    deg = degp_ref[0] + degp_ref[1] + 1.0
    dinv = lax.rsqrt(deg)
    z = dinv[:, None] * (aggp_ref[0] + aggp_ref[1] + hs2_ref[...]) + b2_ref[...]
    m = jnp.max(z, axis=1, keepdims=True)
    lse = jnp.log(jnp.sum(jnp.exp(z - m), axis=1, keepdims=True)) + m
    out_ref[...] = z - lse


def kernel(x, edge_index, W1, b1, W2, b2):
    n, d_in = x.shape
    e = edge_index.shape[1]
    d_hid = W1.shape[1]

    n_pad = _round_up(n, _BLK)
    per_w = _round_up((e + NW - 1) // NW, _BLK)
    e_pad = per_w * NW

    src = edge_index[0].astype(jnp.int32)
    dst = edge_index[1].astype(jnp.int32)
    npad_e = e_pad - e
    if npad_e:
        spread = max(n_pad - n, 1)
        pad_idx = n + (jnp.arange(npad_e, dtype=jnp.int32) % spread)
        src = jnp.concatenate([src, pad_idx])
        dst = jnp.concatenate([dst, pad_idx])
    src2d = src.reshape(e_pad // 128, 128)
    dst2d = dst.reshape(e_pad // 128, 128)

    sc_deg = _make_sc_deg(n_pad, per_w)
    sc_mp = _make_sc_msgpass(n_pad, per_w)

    degp = sc_deg(dst2d)

    grid = (n_pad // _BLK,)
    row_spec = pl.BlockSpec((_BLK, d_hid), lambda i: (i, 0))
    degp_spec = pl.BlockSpec((NC, _BLK), lambda i: (0, i))
    aggp_spec = pl.BlockSpec((NC, _BLK, d_hid), lambda i: (0, i, 0))
    bias_spec = pl.BlockSpec((1, d_hid), lambda i: (0, 0))

    hs1 = pl.pallas_call(
        _tc_a_body,
        grid=grid,
        in_specs=[
            pl.BlockSpec((_BLK, d_in), lambda i: (i, 0)),
            pl.BlockSpec((d_in, d_hid), lambda i: (0, 0)),
            degp_spec,
        ],
        out_specs=row_spec,
        out_shape=jax.ShapeDtypeStruct((n_pad, d_hid), jnp.float32),
    )(x, W1, degp)

    agg1 = sc_mp(hs1, src2d, dst2d)

    hs2 = pl.pallas_call(
        _tc_b_body,
        grid=grid,
        in_specs=[
            aggp_spec,
            row_spec,
            degp_spec,
            bias_spec,
            pl.BlockSpec((d_hid, d_hid), lambda i: (0, 0)),
        ],
        out_specs=row_spec,
        out_shape=jax.ShapeDtypeStruct((n_pad, d_hid), jnp.float32),
    )(agg1, hs1, degp, b1.reshape(1, d_hid), W2)

    agg2 = sc_mp(hs2, src2d, dst2d)

    out = pl.pallas_call(
        _tc_c_body,
        grid=grid,
        in_specs=[aggp_spec, row_spec, degp_spec, bias_spec],
        out_specs=row_spec,
        out_shape=jax.ShapeDtypeStruct((n_pad, d_hid), jnp.float32),
    )(agg2, hs2, degp, b2.reshape(1, d_hid))

    return out[:n]
